# baseline (speedup 1.0000x reference)
import jax
import jax.numpy as jnp
from jax import lax
from jax.experimental import pallas as pl
from jax.experimental.pallas import tpu as pltpu
from jax.experimental.pallas import tpu_sc as plsc

S = 2048
D = 32
MAX_REL = 128
NC, NS = 2, 16
WQ = 31
BQ = 14
OUT5 = (S, 4, 16, 8, 128)


def _sc_body(tk, out_k, tbl, win, sb, sem):
    wid = lax.axis_index("s") * NC + lax.axis_index("c")
    h = wid % 2
    G = wid // 2
    lane = lax.iota(jnp.int32, 16)

    pltpu.async_copy(tk, tbl, sem).wait()

    for dtl in range(2):
        for ds_ in range(8):
            dvec = jnp.full((16,), 16 * h + 8 * dtl + ds_, jnp.int32)
            v0 = plsc.load_gather(tbl, [jnp.zeros((16,), jnp.int32), dvec])
            v1 = plsc.load_gather(
                tbl, [jnp.full((16,), 2 * MAX_REL, jnp.int32), dvec]
            )

            def cfill(n, _, dtl=dtl, ds_=ds_, v0=v0, v1=v1):
                q = n // 8
                lc = n % 8
                win[dtl, q, ds_, pl.ds(16 * lc, 16)] = v0
                win[dtl, q + 17, ds_, pl.ds(16 * lc, 16)] = v1
                return _
            lax.fori_loop(0, BQ * 8, cfill, None)

    def bfill(dst, qoff, a):
        def one(n, _):
            dtl = n // 192
            r = n % 192
            qb = r // 64
            ds_ = (r % 64) // 8
            lc = r % 8
            c0 = (127 - a) + 128 * (BQ + qb) + 16 * lc
            tidx = jnp.clip(c0 + lane - (S - 1), -MAX_REL, MAX_REL) + MAX_REL
            dvec = jnp.full((16,), 16 * h + 8 * dtl + ds_, jnp.int32)
            dst[dtl, qoff + qb, ds_, pl.ds(16 * lc, 16)] = plsc.load_gather(
                tbl, [tidx, dvec]
            )
            return _
        lax.fori_loop(0, 2 * 3 * 64, one, None)

    bfill(win, BQ, G)

    def do_class(j, _):
        a = G + 16 * j

        def fire(t, _):
            i = a + 128 * t
            pltpu.make_async_copy(
                win.at[:, pl.ds(15 - t, 16)],
                out_k.at[i, pl.ds(2 * h, 2)],
                sem,
            ).start()
            return _
        lax.fori_loop(0, 16, fire, None)

        @pl.when(j < 7)
        def _stage():
            bfill(sb, 0, a + 16)

        def drain(t, _):
            pltpu.make_async_copy(
                win.at[:, pl.ds(0, 16)],
                out_k.at[a, pl.ds(2 * h, 2)],
                sem,
            ).wait()
            return _
        lax.fori_loop(0, 16, drain, None)

        @pl.when(j < 7)
        def _commit():
            def cp(n, _):
                dtl = n // 24
                r = n % 24
                qb = r // 8
                lc = r % 8

                def cps(ds_, _, dtl=dtl, qb=qb, lc=lc):
                    win[dtl, BQ + qb, ds_, pl.ds(16 * lc, 16)] = sb[
                        dtl, qb, ds_, pl.ds(16 * lc, 16)
                    ]
                    return _
                lax.fori_loop(0, 8, cps, None)
                return _
            lax.fori_loop(0, 2 * 3 * 8, cp, None)
        return _
    lax.fori_loop(0, 8, do_class, None)


def _sc_kernel(rel_k_table):
    f = pl.kernel(
        _sc_body,
        out_type=(jax.ShapeDtypeStruct(OUT5, jnp.float32),),
        mesh=plsc.VectorSubcoreMesh(core_axis_name="c", subcore_axis_name="s"),
        compiler_params=pltpu.CompilerParams(
            use_tc_tiling_on_sc=False, needs_layout_passes=False
        ),
        scratch_types=[
            pltpu.VMEM((2 * MAX_REL + 1, D), jnp.float32),
            pltpu.VMEM((2, WQ, 8, 128), jnp.float32),
            pltpu.VMEM((2, 3, 8, 128), jnp.float32),
            pltpu.SemaphoreType.DMA,
        ],
    )
    return f(rel_k_table)[0]


def _tc_body(tvt_ref, out_ref, bt, bands, win):
    a = pl.program_id(0)
    t = pl.program_id(1)

    @pl.when((a == 0) & (t == 0))
    def _init():
        bt[:, pl.ds(0, 128)] = jnp.broadcast_to(
            tvt_ref[:, pl.ds(0, 1)], (32, 128)
        )
        bt[:, pl.ds(128, 255)] = tvt_ref[:, pl.ds(1, 255)]
        bt[:, pl.ds(383, 129)] = jnp.broadcast_to(
            tvt_ref[:, pl.ds(2 * MAX_REL, 1)], (32, 129)
        )
        for o in range(384):
            for dt in range(4):
                bands[o, dt] = bt[pl.ds(8 * dt, 8), pl.ds(o, 128)]
        for dt in range(4):
            c0 = jnp.broadcast_to(bt[pl.ds(8 * dt, 8), pl.ds(0, 1)], (8, 128))
            c1 = jnp.broadcast_to(
                bt[pl.ds(8 * dt, 8), pl.ds(511, 1)], (8, 128)
            )
            win[dt, pl.ds(0, BQ)] = jnp.broadcast_to(c0[None], (BQ, 8, 128))
            win[dt, pl.ds(BQ + 3, WQ - BQ - 3)] = jnp.broadcast_to(
                c1[None], (WQ - BQ - 3, 8, 128)
            )

    @pl.when(t == 0)
    def _band():
        for qb in range(3):
            o = (127 + 128 * qb) - a
            for dt in range(4):
                win[dt, BQ + qb] = bands[o, dt]

    out_ref[0] = win[:, pl.ds(15 - t, 16)]


def _tc_kernel(rel_v_table):
    tvt = rel_v_table.T
    return pl.pallas_call(
        _tc_body,
        grid=(128, 16),
        out_shape=jax.ShapeDtypeStruct(OUT5, jnp.float32),
        in_specs=[pl.BlockSpec((D, 257), lambda a, t: (0, 0))],
        out_specs=pl.BlockSpec(
            (1, 4, 16, 8, 128), lambda a, t: (a + 128 * t, 0, 0, 0, 0)
        ),
        scratch_shapes=[
            pltpu.VMEM((32, 512), jnp.float32),
            pltpu.VMEM((384, 4, 8, 128), jnp.float32),
            pltpu.VMEM((4, WQ, 8, 128), jnp.float32),
        ],
    )(tvt)


def kernel(seq_len, rel_k_table, rel_v_table):
    del seq_len
    ok = _sc_kernel(rel_k_table)
    ov = _tc_kernel(rel_v_table)
    ok = ok.transpose(0, 2, 4, 1, 3).reshape(S, S, D)
    ov = ov.transpose(0, 2, 4, 1, 3).reshape(S, S, D)
    return (ok, ov)


# speedup vs baseline: 2.4806x; 2.4806x over previous
import jax
import jax.numpy as jnp
from jax import lax
from jax.experimental import pallas as pl
from jax.experimental.pallas import tpu as pltpu
from jax.experimental.pallas import tpu_sc as plsc

S = 2048
D = 32
MAX_REL = 128
NC, NS = 2, 16
WQ = 31
BQ = 14
OUT5 = (S, 4, 16, 8, 128)


def _sc_body(tk, out_k, tbl, win, sb, sem):
    wid = lax.axis_index("s") * NC + lax.axis_index("c")
    h = wid % 2
    G = wid // 2
    lane = lax.iota(jnp.int32, 16)

    pltpu.async_copy(tk, tbl, sem).wait()

    for dtl in range(2):
        for ds_ in range(8):
            dvec = jnp.full((16,), 16 * h + 8 * dtl + ds_, jnp.int32)
            v0 = plsc.load_gather(tbl, [jnp.zeros((16,), jnp.int32), dvec])
            v1 = plsc.load_gather(
                tbl, [jnp.full((16,), 2 * MAX_REL, jnp.int32), dvec]
            )

            def cfill(n, _, dtl=dtl, ds_=ds_, v0=v0, v1=v1):
                q = n // 8
                lc = n % 8
                win[dtl, q, ds_, pl.ds(16 * lc, 16)] = v0
                win[dtl, q + 17, ds_, pl.ds(16 * lc, 16)] = v1
                return _
            lax.fori_loop(0, BQ * 8, cfill, None)

    def bfill(dst, qoff, a):
        def one(n, _):
            dtl = n // 192
            r = n % 192
            qb = r // 64
            ds_ = (r % 64) // 8
            lc = r % 8
            c0 = (127 - a) + 128 * (BQ + qb) + 16 * lc
            tidx = jnp.clip(c0 + lane - (S - 1), -MAX_REL, MAX_REL) + MAX_REL
            dvec = jnp.full((16,), 16 * h + 8 * dtl + ds_, jnp.int32)
            dst[dtl, qoff + qb, ds_, pl.ds(16 * lc, 16)] = plsc.load_gather(
                tbl, [tidx, dvec]
            )
            return _
        lax.fori_loop(0, 2 * 3 * 64, one, None)

    bfill(win, BQ, G)

    def do_class(j, _):
        a = G + 16 * j

        def fire(t, _):
            i = a + 128 * t
            pltpu.make_async_copy(
                win.at[:, pl.ds(15 - t, 16)],
                out_k.at[i, pl.ds(2 * h, 2)],
                sem,
            ).start()
            return _
        lax.fori_loop(0, 16, fire, None)

        @pl.when(j < 7)
        def _stage():
            bfill(sb, 0, a + 16)

        def drain(t, _):
            pltpu.make_async_copy(
                win.at[:, pl.ds(0, 16)],
                out_k.at[a, pl.ds(2 * h, 2)],
                sem,
            ).wait()
            return _
        lax.fori_loop(0, 16, drain, None)

        @pl.when(j < 7)
        def _commit():
            def cp(n, _):
                dtl = n // 24
                r = n % 24
                qb = r // 8
                lc = r % 8

                def cps(ds_, _, dtl=dtl, qb=qb, lc=lc):
                    win[dtl, BQ + qb, ds_, pl.ds(16 * lc, 16)] = sb[
                        dtl, qb, ds_, pl.ds(16 * lc, 16)
                    ]
                    return _
                lax.fori_loop(0, 8, cps, None)
                return _
            lax.fori_loop(0, 2 * 3 * 8, cp, None)
        return _
    lax.fori_loop(0, 8, do_class, None)


def _sc_kernel(rel_k_table):
    f = pl.kernel(
        _sc_body,
        out_type=(jax.ShapeDtypeStruct(OUT5, jnp.float32),),
        mesh=plsc.VectorSubcoreMesh(core_axis_name="c", subcore_axis_name="s"),
        compiler_params=pltpu.CompilerParams(
            use_tc_tiling_on_sc=False, needs_layout_passes=False
        ),
        scratch_types=[
            pltpu.VMEM((2 * MAX_REL + 1, D), jnp.float32),
            pltpu.VMEM((2, WQ, 8, 128), jnp.float32),
            pltpu.VMEM((2, 3, 8, 128), jnp.float32),
            pltpu.SemaphoreType.DMA,
        ],
    )
    return f(rel_k_table)[0]


def _tc_body(tvt_ref, out_ref, bt, bands, win, sem):
    bt[:, pl.ds(0, 128)] = jnp.broadcast_to(tvt_ref[:, pl.ds(0, 1)], (32, 128))
    bt[:, pl.ds(128, 255)] = tvt_ref[:, pl.ds(1, 255)]
    bt[:, pl.ds(383, 129)] = jnp.broadcast_to(
        tvt_ref[:, pl.ds(2 * MAX_REL, 1)], (32, 129)
    )

    for o in range(384):
        for dt in range(4):
            bands[o, dt] = bt[pl.ds(8 * dt, 8), pl.ds(o, 128)]

    for p in range(2):
        for dt in range(4):
            c0 = jnp.broadcast_to(bt[pl.ds(8 * dt, 8), pl.ds(0, 1)], (8, 128))
            c1 = jnp.broadcast_to(bt[pl.ds(8 * dt, 8), pl.ds(511, 1)], (8, 128))
            win[p, dt, pl.ds(0, BQ)] = jnp.broadcast_to(c0[None], (BQ, 8, 128))
            win[p, dt, pl.ds(BQ + 3, WQ - BQ - 3)] = jnp.broadcast_to(
                c1[None], (WQ - BQ - 3, 8, 128)
            )

    def band(a, p):
        for qb in range(3):
            o = (127 + 128 * qb) - a
            for dt in range(4):
                win[p, dt, BQ + qb] = bands[o, dt]

    band(0, 0)

    def do_class(a, _):
        p = a % 2

        def fire(t, _):
            i = a + 128 * t
            pltpu.make_async_copy(
                win.at[p, :, pl.ds(15 - t, 16)], out_ref.at[i], sem
            ).start()
            return _
        lax.fori_loop(0, 16, fire, None)

        @pl.when(a >= 1)
        def _drain_prev():
            def drain(t, _):
                pltpu.make_async_copy(
                    win.at[0, :, pl.ds(0, 16)], out_ref.at[a], sem
                ).wait()
                return _
            lax.fori_loop(0, 16, drain, None)

        @pl.when(a < 127)
        def _stage_next():
            band(a + 1, 1 - p)
        return _
    lax.fori_loop(0, 128, do_class, None)

    def drain_last(t, _):
        pltpu.make_async_copy(
            win.at[0, :, pl.ds(0, 16)], out_ref.at[0], sem
        ).wait()
        return _
    lax.fori_loop(0, 16, drain_last, None)


def _tc_kernel(rel_v_table):
    tvt = rel_v_table.T
    return pl.pallas_call(
        _tc_body,
        out_shape=jax.ShapeDtypeStruct(OUT5, jnp.float32),
        in_specs=[pl.BlockSpec(memory_space=pltpu.VMEM)],
        out_specs=pl.BlockSpec(memory_space=pl.ANY),
        scratch_shapes=[
            pltpu.VMEM((32, 512), jnp.float32),
            pltpu.VMEM((384, 4, 8, 128), jnp.float32),
            pltpu.VMEM((2, 4, WQ, 8, 128), jnp.float32),
            pltpu.SemaphoreType.DMA,
        ],
    )(tvt)


def kernel(seq_len, rel_k_table, rel_v_table):
    del seq_len
    ok = _sc_kernel(rel_k_table)
    ov = _tc_kernel(rel_v_table)
    ok = ok.transpose(0, 2, 4, 1, 3).reshape(S, S, D)
    ov = ov.transpose(0, 2, 4, 1, 3).reshape(S, S, D)
    return (ok, ov)
